# baseline (speedup 1.0000x reference)
import jax
import jax.numpy as jnp
from jax import lax
from jax.experimental import pallas as pl
from jax.experimental.pallas import tpu as pltpu
from jax.experimental.pallas import tpu_sc as plsc

N, E, D = 10000, 320000, 128
NC, NS = 2, 16
NW = NC * NS
EPT = E // NW
CH = 112
NCH = 91
EPP = NCH * CH
DEGW = 16
NP = 10240
RPS = NP // NS
ZR = 128
LANES = 16

_SC_PARAMS = pltpu.CompilerParams(use_tc_tiling_on_sc=False)


def _zero_rows(ref, nrows, ncols):
    z16 = jnp.zeros((LANES,), jnp.float32)

    def row(r, carry):
        for c in range(ncols // LANES):
            ref[r, pl.ds(c * LANES, LANES)] = z16
        return carry

    lax.fori_loop(0, nrows, row, 0)


def _worker_id():
    return lax.axis_index("s") * NC + lax.axis_index("c")


def _sc_agg_body(h_hbm, src_hbm, dst_hbm, agg_hbm,
                 src_v, dst_v, r0_v, r1_v, acc_sh, sem0, sem1):
    cid = lax.axis_index("c")
    sid = lax.axis_index("s")
    wid = _worker_id()

    pltpu.sync_copy(src_hbm.at[wid], src_v)
    pltpu.sync_copy(dst_hbm.at[wid], dst_v)

    _zero_rows(r0_v, CH, D)
    base = sid * RPS
    for k in range(RPS // CH):
        pltpu.sync_copy(r0_v, acc_sh.at[pl.ds(base + k * CH, CH)])
    rem = RPS - (RPS // CH) * CH
    if rem:
        pltpu.sync_copy(r0_v.at[pl.ds(0, rem)],
                        acc_sh.at[pl.ds(base + RPS - rem, rem)])

    plsc.subcore_barrier()

    pltpu.async_copy(h_hbm.at[src_v.at[0]], r0_v, sem0)

    def pair(k, carry):
        c0 = 2 * k
        pltpu.make_async_copy(h_hbm.at[src_v.at[c0]], r0_v, sem0).wait()
        pltpu.async_copy(h_hbm.at[src_v.at[c0 + 1]], r1_v, sem1)
        pltpu.sync_copy(r0_v, acc_sh.at[dst_v.at[c0]], add=True)
        pltpu.make_async_copy(h_hbm.at[src_v.at[c0 + 1]], r1_v, sem1).wait()
        pltpu.async_copy(h_hbm.at[src_v.at[c0 + 2]], r0_v, sem0)
        pltpu.sync_copy(r1_v, acc_sh.at[dst_v.at[c0 + 1]], add=True)
        return carry

    lax.fori_loop(0, (NCH - 1) // 2, pair, 0)
    pltpu.make_async_copy(h_hbm.at[src_v.at[NCH - 1]], r0_v, sem0).wait()
    pltpu.sync_copy(r0_v, acc_sh.at[dst_v.at[NCH - 1]], add=True)

    plsc.subcore_barrier()

    pltpu.sync_copy(acc_sh.at[pl.ds(base, RPS)],
                    agg_hbm.at[cid, pl.ds(base, RPS)])


_sc_agg = pl.kernel(
    _sc_agg_body,
    out_type=jax.ShapeDtypeStruct((NC, NP, D), jnp.float32),
    mesh=plsc.VectorSubcoreMesh(core_axis_name="c", subcore_axis_name="s"),
    scratch_types=(
        pltpu.VMEM((NCH, CH), jnp.int32),
        pltpu.VMEM((NCH, CH), jnp.int32),
        pltpu.VMEM((CH, D), jnp.float32),
        pltpu.VMEM((CH, D), jnp.float32),
        pltpu.VMEM_SHARED((NP, D), jnp.float32),
        pltpu.SemaphoreType.DMA,
        pltpu.SemaphoreType.DMA,
    ),
    compiler_params=_SC_PARAMS,
)


def _sc_deg_body(dst_hbm, deg_hbm, dst_v, zd_v, ones_v, dacc_sh):
    cid = lax.axis_index("c")
    sid = lax.axis_index("s")
    wid = _worker_id()

    pltpu.sync_copy(dst_hbm.at[wid], dst_v)

    _zero_rows(zd_v, RPS, DEGW)
    base = sid * RPS
    pltpu.sync_copy(zd_v, dacc_sh.at[pl.ds(base, RPS)])

    one16 = jnp.ones((LANES,), jnp.float32)

    def orow(r, carry):
        ones_v[r, pl.ds(0, LANES)] = one16
        return carry

    lax.fori_loop(0, CH, orow, 0)

    plsc.subcore_barrier()

    def chunk(c, carry):
        pltpu.sync_copy(ones_v, dacc_sh.at[dst_v.at[c]], add=True)
        return carry

    lax.fori_loop(0, NCH, chunk, 0)

    plsc.subcore_barrier()

    pltpu.sync_copy(dacc_sh.at[pl.ds(base, RPS)],
                    deg_hbm.at[cid, pl.ds(base, RPS)])


_sc_deg = pl.kernel(
    _sc_deg_body,
    out_type=jax.ShapeDtypeStruct((NC, NP, DEGW), jnp.float32),
    mesh=plsc.VectorSubcoreMesh(core_axis_name="c", subcore_axis_name="s"),
    scratch_types=(
        pltpu.VMEM((NCH, CH), jnp.int32),
        pltpu.VMEM((RPS, DEGW), jnp.float32),
        pltpu.VMEM((CH, DEGW), jnp.float32),
        pltpu.VMEM_SHARED((NP, DEGW), jnp.float32),
    ),
    compiler_params=_SC_PARAMS,
)

BN = 1000


def _dense0_body(aggp_ref, degp_ref, w_ref, b_ref, h_ref, degb_ref):
    p = aggp_ref[...]
    agg = p[0] + p[1]
    d = degp_ref[...]
    deg = (jnp.sum(d[0], axis=1) + jnp.sum(d[1], axis=1)) * (1.0 / DEGW)
    deg = jnp.maximum(deg, 1.0)[:, None]
    x = agg / deg
    y = jnp.dot(x, w_ref[...], preferred_element_type=jnp.float32)
    h_ref[...] = jnp.maximum(y + b_ref[...], 0.0)
    degb_ref[...] = jnp.broadcast_to(deg, (BN, D))


def _dense0(aggp, degp, w, b):
    return pl.pallas_call(
        _dense0_body,
        grid=(N // BN,),
        in_specs=[
            pl.BlockSpec((NC, BN, D), lambda i: (0, i, 0)),
            pl.BlockSpec((NC, BN, DEGW), lambda i: (0, i, 0)),
            pl.BlockSpec((D, D), lambda i: (0, 0)),
            pl.BlockSpec((1, D), lambda i: (0, 0)),
        ],
        out_specs=[
            pl.BlockSpec((BN, D), lambda i: (i, 0)),
            pl.BlockSpec((BN, D), lambda i: (i, 0)),
        ],
        out_shape=[
            jax.ShapeDtypeStruct((N, D), jnp.float32),
            jax.ShapeDtypeStruct((N, D), jnp.float32),
        ],
    )(aggp, degp, w, b)


def _dense_body(aggp_ref, degb_ref, w_ref, b_ref, h_ref):
    p = aggp_ref[...]
    x = (p[0] + p[1]) / degb_ref[...]
    y = jnp.dot(x, w_ref[...], preferred_element_type=jnp.float32)
    h_ref[...] = jnp.maximum(y + b_ref[...], 0.0)


def _dense(aggp, degb, w, b):
    return pl.pallas_call(
        _dense_body,
        grid=(N // BN,),
        in_specs=[
            pl.BlockSpec((NC, BN, D), lambda i: (0, i, 0)),
            pl.BlockSpec((BN, D), lambda i: (i, 0)),
            pl.BlockSpec((D, D), lambda i: (0, 0)),
            pl.BlockSpec((1, D), lambda i: (0, 0)),
        ],
        out_specs=pl.BlockSpec((BN, D), lambda i: (i, 0)),
        out_shape=jax.ShapeDtypeStruct((N, D), jnp.float32),
    )(aggp, degb, w, b)


def kernel(h, edge_index, W0, b0, W1, b1, W2, b2):
    pad = ((0, 0), (0, EPP - EPT))
    src = jnp.pad(edge_index[0].reshape(NW, EPT), pad,
                  constant_values=0).reshape(NW, NCH, CH)
    dst = jnp.pad(edge_index[1].reshape(NW, EPT), pad,
                  constant_values=N).reshape(NW, NCH, CH)
    degp = _sc_deg(dst)
    aggp = _sc_agg(h, src, dst)
    h1, degb = _dense0(aggp, degp, W0, b0.reshape(1, D))
    aggp = _sc_agg(h1, src, dst)
    h2 = _dense(aggp, degb, W1, b1.reshape(1, D))
    aggp = _sc_agg(h2, src, dst)
    return _dense(aggp, degb, W2, b2.reshape(1, D))


# speedup vs baseline: 2.1214x; 2.1214x over previous
import jax
import jax.numpy as jnp
from jax import lax
from jax.experimental import pallas as pl
from jax.experimental.pallas import tpu as pltpu
from jax.experimental.pallas import tpu_sc as plsc

N, E, D = 10000, 320000, 128
NC, NS = 2, 16
NW = NC * NS
EPT = E // NW
CH = 80
NCH = 125
EPP = NCH * CH
DEGW = 16
NP = 10240
RPS = NP // NS
ZR = 128
LANES = 16

_SC_PARAMS = pltpu.CompilerParams(use_tc_tiling_on_sc=False)


def _zero_rows(ref, nrows, ncols):
    z16 = jnp.zeros((LANES,), jnp.float32)

    def row(r, carry):
        for c in range(ncols // LANES):
            ref[r, pl.ds(c * LANES, LANES)] = z16
        return carry

    lax.fori_loop(0, nrows, row, 0)


def _worker_id():
    return lax.axis_index("s") * NC + lax.axis_index("c")


def _sc_agg_body(h_hbm, src_hbm, dst_hbm, agg_hbm,
                 src_v, dst_v, r0_v, r1_v, acc_sh, sem0, sem1):
    cid = lax.axis_index("c")
    sid = lax.axis_index("s")
    wid = _worker_id()

    pltpu.sync_copy(src_hbm.at[wid], src_v)
    pltpu.sync_copy(dst_hbm.at[wid], dst_v)

    _zero_rows(r0_v, CH, D)
    base = sid * RPS
    for k in range(RPS // CH):
        pltpu.sync_copy(r0_v, acc_sh.at[pl.ds(base + k * CH, CH)])
    rem = RPS - (RPS // CH) * CH
    if rem:
        pltpu.sync_copy(r0_v.at[pl.ds(0, rem)],
                        acc_sh.at[pl.ds(base + RPS - rem, rem)])

    plsc.subcore_barrier()

    pltpu.async_copy(h_hbm.at[src_v.at[0]], r0_v, sem0)

    def pair(k, carry):
        c0 = 2 * k
        pltpu.make_async_copy(h_hbm.at[src_v.at[c0]], r0_v, sem0).wait()
        pltpu.async_copy(h_hbm.at[src_v.at[c0 + 1]], r1_v, sem1)
        pltpu.sync_copy(r0_v, acc_sh.at[dst_v.at[c0]], add=True)
        pltpu.make_async_copy(h_hbm.at[src_v.at[c0 + 1]], r1_v, sem1).wait()
        pltpu.async_copy(h_hbm.at[src_v.at[c0 + 2]], r0_v, sem0)
        pltpu.sync_copy(r1_v, acc_sh.at[dst_v.at[c0 + 1]], add=True)
        return carry

    lax.fori_loop(0, (NCH - 1) // 2, pair, 0)
    pltpu.make_async_copy(h_hbm.at[src_v.at[NCH - 1]], r0_v, sem0).wait()
    pltpu.sync_copy(r0_v, acc_sh.at[dst_v.at[NCH - 1]], add=True)

    plsc.subcore_barrier()

    pltpu.sync_copy(acc_sh.at[pl.ds(base, RPS)],
                    agg_hbm.at[cid, pl.ds(base, RPS)])


_sc_agg = pl.kernel(
    _sc_agg_body,
    out_type=jax.ShapeDtypeStruct((NC, NP, D), jnp.float32),
    mesh=plsc.VectorSubcoreMesh(core_axis_name="c", subcore_axis_name="s"),
    scratch_types=(
        pltpu.VMEM((NCH, CH), jnp.int32),
        pltpu.VMEM((NCH, CH), jnp.int32),
        pltpu.VMEM((CH, D), jnp.float32),
        pltpu.VMEM((CH, D), jnp.float32),
        pltpu.VMEM_SHARED((NP, D), jnp.float32),
        pltpu.SemaphoreType.DMA,
        pltpu.SemaphoreType.DMA,
    ),
    compiler_params=_SC_PARAMS,
)


def _sc_deg_body(dst_hbm, deg_hbm, dst_v, zd_v, ones_v, dacc_sh):
    cid = lax.axis_index("c")
    sid = lax.axis_index("s")
    wid = _worker_id()

    pltpu.sync_copy(dst_hbm.at[wid], dst_v)

    _zero_rows(zd_v, RPS, DEGW)
    base = sid * RPS
    pltpu.sync_copy(zd_v, dacc_sh.at[pl.ds(base, RPS)])

    one16 = jnp.ones((LANES,), jnp.float32)

    def orow(r, carry):
        ones_v[r, pl.ds(0, LANES)] = one16
        return carry

    lax.fori_loop(0, CH, orow, 0)

    plsc.subcore_barrier()

    def chunk(c, carry):
        pltpu.sync_copy(ones_v, dacc_sh.at[dst_v.at[c]], add=True)
        return carry

    lax.fori_loop(0, NCH, chunk, 0)

    plsc.subcore_barrier()

    pltpu.sync_copy(dacc_sh.at[pl.ds(base, RPS)],
                    deg_hbm.at[cid, pl.ds(base, RPS)])


_sc_deg = pl.kernel(
    _sc_deg_body,
    out_type=jax.ShapeDtypeStruct((NC, NP, DEGW), jnp.float32),
    mesh=plsc.VectorSubcoreMesh(core_axis_name="c", subcore_axis_name="s"),
    scratch_types=(
        pltpu.VMEM((NCH, CH), jnp.int32),
        pltpu.VMEM((RPS, DEGW), jnp.float32),
        pltpu.VMEM((CH, DEGW), jnp.float32),
        pltpu.VMEM_SHARED((NP, DEGW), jnp.float32),
    ),
    compiler_params=_SC_PARAMS,
)

BN = 1000


def _dense0_body(aggp_ref, degp_ref, w_ref, b_ref, h_ref, degb_ref):
    p = aggp_ref[...]
    agg = p[0] + p[1]
    d = degp_ref[...]
    deg = (jnp.sum(d[0], axis=1) + jnp.sum(d[1], axis=1)) * (1.0 / DEGW)
    deg = jnp.maximum(deg, 1.0)[:, None]
    x = agg / deg
    y = jnp.dot(x, w_ref[...], preferred_element_type=jnp.float32)
    h_ref[...] = jnp.maximum(y + b_ref[...], 0.0)
    degb_ref[...] = jnp.broadcast_to(deg, (BN, D))


def _dense0(aggp, degp, w, b):
    return pl.pallas_call(
        _dense0_body,
        grid=(N // BN,),
        in_specs=[
            pl.BlockSpec((NC, BN, D), lambda i: (0, i, 0)),
            pl.BlockSpec((NC, BN, DEGW), lambda i: (0, i, 0)),
            pl.BlockSpec((D, D), lambda i: (0, 0)),
            pl.BlockSpec((1, D), lambda i: (0, 0)),
        ],
        out_specs=[
            pl.BlockSpec((BN, D), lambda i: (i, 0)),
            pl.BlockSpec((BN, D), lambda i: (i, 0)),
        ],
        out_shape=[
            jax.ShapeDtypeStruct((N, D), jnp.float32),
            jax.ShapeDtypeStruct((N, D), jnp.float32),
        ],
    )(aggp, degp, w, b)


def _dense_body(aggp_ref, degb_ref, w_ref, b_ref, h_ref):
    p = aggp_ref[...]
    x = (p[0] + p[1]) / degb_ref[...]
    y = jnp.dot(x, w_ref[...], preferred_element_type=jnp.float32)
    h_ref[...] = jnp.maximum(y + b_ref[...], 0.0)


def _dense(aggp, degb, w, b):
    return pl.pallas_call(
        _dense_body,
        grid=(N // BN,),
        in_specs=[
            pl.BlockSpec((NC, BN, D), lambda i: (0, i, 0)),
            pl.BlockSpec((BN, D), lambda i: (i, 0)),
            pl.BlockSpec((D, D), lambda i: (0, 0)),
            pl.BlockSpec((1, D), lambda i: (0, 0)),
        ],
        out_specs=pl.BlockSpec((BN, D), lambda i: (i, 0)),
        out_shape=jax.ShapeDtypeStruct((N, D), jnp.float32),
    )(aggp, degb, w, b)


def kernel(h, edge_index, W0, b0, W1, b1, W2, b2):
    pad = ((0, 0), (0, EPP - EPT))
    src = jnp.pad(edge_index[0].reshape(NW, EPT), pad,
                  constant_values=0).reshape(NW, NCH, CH)
    dst = jnp.pad(edge_index[1].reshape(NW, EPT), pad,
                  constant_values=N).reshape(NW, NCH, CH)
    degp = _sc_deg(dst)
    aggp = _sc_agg(h, src, dst)
    h1, degb = _dense0(aggp, degp, W0, b0.reshape(1, D))
    aggp = _sc_agg(h1, src, dst)
    h2 = _dense(aggp, degb, W1, b1.reshape(1, D))
    aggp = _sc_agg(h2, src, dst)
    return _dense(aggp, degb, W2, b2.reshape(1, D))


# speedup vs baseline: 2.3399x; 1.1030x over previous
import jax
import jax.numpy as jnp
from jax import lax
from jax.experimental import pallas as pl
from jax.experimental.pallas import tpu as pltpu
from jax.experimental.pallas import tpu_sc as plsc

N, E, D = 10000, 320000, 128
NC, NS = 2, 16
NW = NC * NS
EPT = E // NW
CH = 80
NCH = 125
EPP = NCH * CH
DEGW = 16
NP = 10240
RPS = NP // NS
ZR = 128
LANES = 16

_SC_PARAMS = pltpu.CompilerParams(use_tc_tiling_on_sc=False)


def _zero_rows(ref, nrows, ncols, dtype=jnp.float32):
    width = 2 * LANES if dtype == jnp.bfloat16 else LANES
    zv = jnp.zeros((width,), dtype)

    def row(r, carry):
        for c in range(ncols // width):
            ref[r, pl.ds(c * width, width)] = zv
        return carry

    lax.fori_loop(0, nrows, row, 0)


def _worker_id():
    return lax.axis_index("s") * NC + lax.axis_index("c")


def _sc_agg_body(h_hbm, src_hbm, dst_hbm, agg_hbm,
                 src_v, dst_v, r0_v, r1_v, acc_sh, sem0, sem1):
    cid = lax.axis_index("c")
    sid = lax.axis_index("s")
    wid = _worker_id()

    pltpu.sync_copy(src_hbm.at[wid], src_v)
    pltpu.sync_copy(dst_hbm.at[wid], dst_v)

    _zero_rows(r0_v, CH, D, jnp.bfloat16)
    base = sid * RPS
    for k in range(RPS // CH):
        pltpu.sync_copy(r0_v, acc_sh.at[pl.ds(base + k * CH, CH)])
    rem = RPS - (RPS // CH) * CH
    if rem:
        pltpu.sync_copy(r0_v.at[pl.ds(0, rem)],
                        acc_sh.at[pl.ds(base + RPS - rem, rem)])

    plsc.subcore_barrier()

    pltpu.async_copy(h_hbm.at[src_v.at[0]], r0_v, sem0)

    def pair(k, carry):
        c0 = 2 * k
        pltpu.make_async_copy(h_hbm.at[src_v.at[c0]], r0_v, sem0).wait()
        pltpu.async_copy(h_hbm.at[src_v.at[c0 + 1]], r1_v, sem1)
        pltpu.sync_copy(r0_v, acc_sh.at[dst_v.at[c0]], add=True)
        pltpu.make_async_copy(h_hbm.at[src_v.at[c0 + 1]], r1_v, sem1).wait()
        pltpu.async_copy(h_hbm.at[src_v.at[c0 + 2]], r0_v, sem0)
        pltpu.sync_copy(r1_v, acc_sh.at[dst_v.at[c0 + 1]], add=True)
        return carry

    lax.fori_loop(0, (NCH - 1) // 2, pair, 0)
    pltpu.make_async_copy(h_hbm.at[src_v.at[NCH - 1]], r0_v, sem0).wait()
    pltpu.sync_copy(r0_v, acc_sh.at[dst_v.at[NCH - 1]], add=True)

    plsc.subcore_barrier()

    pltpu.sync_copy(acc_sh.at[pl.ds(base, RPS)],
                    agg_hbm.at[cid, pl.ds(base, RPS)])


_sc_agg = pl.kernel(
    _sc_agg_body,
    out_type=jax.ShapeDtypeStruct((NC, NP, D), jnp.bfloat16),
    mesh=plsc.VectorSubcoreMesh(core_axis_name="c", subcore_axis_name="s"),
    scratch_types=(
        pltpu.VMEM((NCH, CH), jnp.int32),
        pltpu.VMEM((NCH, CH), jnp.int32),
        pltpu.VMEM((CH, D), jnp.bfloat16),
        pltpu.VMEM((CH, D), jnp.bfloat16),
        pltpu.VMEM_SHARED((NP, D), jnp.bfloat16),
        pltpu.SemaphoreType.DMA,
        pltpu.SemaphoreType.DMA,
    ),
    compiler_params=_SC_PARAMS,
)


def _sc_deg_body(dst_hbm, deg_hbm, dst_v, zd_v, ones_v, dacc_sh):
    cid = lax.axis_index("c")
    sid = lax.axis_index("s")
    wid = _worker_id()

    pltpu.sync_copy(dst_hbm.at[wid], dst_v)

    _zero_rows(zd_v, RPS, DEGW)
    base = sid * RPS
    pltpu.sync_copy(zd_v, dacc_sh.at[pl.ds(base, RPS)])

    one16 = jnp.ones((LANES,), jnp.float32)

    def orow(r, carry):
        ones_v[r, pl.ds(0, LANES)] = one16
        return carry

    lax.fori_loop(0, CH, orow, 0)

    plsc.subcore_barrier()

    def chunk(c, carry):
        pltpu.sync_copy(ones_v, dacc_sh.at[dst_v.at[c]], add=True)
        return carry

    lax.fori_loop(0, NCH, chunk, 0)

    plsc.subcore_barrier()

    pltpu.sync_copy(dacc_sh.at[pl.ds(base, RPS)],
                    deg_hbm.at[cid, pl.ds(base, RPS)])


_sc_deg = pl.kernel(
    _sc_deg_body,
    out_type=jax.ShapeDtypeStruct((NC, NP, DEGW), jnp.float32),
    mesh=plsc.VectorSubcoreMesh(core_axis_name="c", subcore_axis_name="s"),
    scratch_types=(
        pltpu.VMEM((NCH, CH), jnp.int32),
        pltpu.VMEM((RPS, DEGW), jnp.float32),
        pltpu.VMEM((CH, DEGW), jnp.float32),
        pltpu.VMEM_SHARED((NP, DEGW), jnp.float32),
    ),
    compiler_params=_SC_PARAMS,
)

BN = 2000


def _dense0_body(aggp_ref, degp_ref, w_ref, b_ref, h_ref, degb_ref):
    p = aggp_ref[...].astype(jnp.float32)
    agg = p[0] + p[1]
    d = degp_ref[...]
    deg = (jnp.sum(d[0], axis=1) + jnp.sum(d[1], axis=1)) * (1.0 / DEGW)
    deg = jnp.maximum(deg, 1.0)[:, None]
    x = agg / deg
    y = jnp.dot(x, w_ref[...], preferred_element_type=jnp.float32)
    h_ref[...] = jnp.maximum(y + b_ref[...], 0.0).astype(h_ref.dtype)
    degb_ref[...] = jnp.broadcast_to(deg, (BN, D))


def _dense0(aggp, degp, w, b):
    return pl.pallas_call(
        _dense0_body,
        grid=(N // BN,),
        in_specs=[
            pl.BlockSpec((NC, BN, D), lambda i: (0, i, 0)),
            pl.BlockSpec((NC, BN, DEGW), lambda i: (0, i, 0)),
            pl.BlockSpec((D, D), lambda i: (0, 0)),
            pl.BlockSpec((1, D), lambda i: (0, 0)),
        ],
        out_specs=[
            pl.BlockSpec((BN, D), lambda i: (i, 0)),
            pl.BlockSpec((BN, D), lambda i: (i, 0)),
        ],
        out_shape=[
            jax.ShapeDtypeStruct((N, D), jnp.bfloat16),
            jax.ShapeDtypeStruct((N, D), jnp.float32),
        ],
    )(aggp, degp, w, b)


def _dense_body(aggp_ref, degb_ref, w_ref, b_ref, h_ref):
    p = aggp_ref[...].astype(jnp.float32)
    x = (p[0] + p[1]) / degb_ref[...]
    y = jnp.dot(x, w_ref[...], preferred_element_type=jnp.float32)
    h_ref[...] = jnp.maximum(y + b_ref[...], 0.0).astype(h_ref.dtype)


def _dense(aggp, degb, w, b, out_dtype):
    return pl.pallas_call(
        _dense_body,
        grid=(N // BN,),
        in_specs=[
            pl.BlockSpec((NC, BN, D), lambda i: (0, i, 0)),
            pl.BlockSpec((BN, D), lambda i: (i, 0)),
            pl.BlockSpec((D, D), lambda i: (0, 0)),
            pl.BlockSpec((1, D), lambda i: (0, 0)),
        ],
        out_specs=pl.BlockSpec((BN, D), lambda i: (i, 0)),
        out_shape=jax.ShapeDtypeStruct((N, D), out_dtype),
    )(aggp, degb, w, b)


def kernel(h, edge_index, W0, b0, W1, b1, W2, b2):
    pad = ((0, 0), (0, EPP - EPT))
    src = jnp.pad(edge_index[0].reshape(NW, EPT), pad,
                  constant_values=0).reshape(NW, NCH, CH)
    dst = jnp.pad(edge_index[1].reshape(NW, EPT), pad,
                  constant_values=N).reshape(NW, NCH, CH)
    degp = _sc_deg(dst)
    aggp = _sc_agg(h.astype(jnp.bfloat16), src, dst)
    h1, degb = _dense0(aggp, degp, W0, b0.reshape(1, D))
    aggp = _sc_agg(h1, src, dst)
    h2 = _dense(aggp, degb, W1, b1.reshape(1, D), jnp.bfloat16)
    aggp = _sc_agg(h2, src, dst)
    return _dense(aggp, degb, W2, b2.reshape(1, D), jnp.float32)
